# Initial kernel scaffold; baseline (speedup 1.0000x reference)
import functools

import jax
import jax.numpy as jnp
from jax.experimental import pallas as pl

_INF = jnp.float32(3.0e38)


def _nn_kernel(unk_ref, kn_ref, idx_ref, w_ref, *, m):
    u = unk_ref[0]
    kn = kn_ref[0]
    d2 = None
    for c in range(3):
        kc = kn[:, c:c + 1]
        uc = u[c].reshape(1, -1)
        diff = kc - uc
        sq = diff * diff
        d2 = sq if d2 is None else d2 + sq
    iota = jax.lax.broadcasted_iota(jnp.int32, d2.shape, 0)
    d = d2
    idxs, dists = [], []
    for _ in range(3):
        mn = jnp.min(d, axis=0, keepdims=True)
        am = jnp.min(jnp.where(d == mn, iota, m), axis=0, keepdims=True)
        d = jnp.where(iota == am, _INF, d)
        idxs.append(am)
        dists.append(jnp.sqrt(jnp.maximum(mn, 0.0)))
    idx_ref[0] = jnp.concatenate(idxs, axis=0)
    dr = [1.0 / (dd + 1e-8) for dd in dists]
    norm = dr[0] + dr[1] + dr[2]
    w_ref[0] = jnp.concatenate([x / norm for x in dr], axis=0)


def _three_nn(unknown_t, known):
    B, _, n = unknown_t.shape
    m = known.shape[1]
    NB = min(n, 2048)
    grid = (B, n // NB)
    return pl.pallas_call(
        functools.partial(_nn_kernel, m=m),
        grid=grid,
        in_specs=[
            pl.BlockSpec((1, 3, NB), lambda b, i: (b, 0, i)),
            pl.BlockSpec((1, m, 3), lambda b, i: (b, 0, 0)),
        ],
        out_specs=[
            pl.BlockSpec((1, 3, NB), lambda b, i: (b, 0, i)),
            pl.BlockSpec((1, 3, NB), lambda b, i: (b, 0, i)),
        ],
        out_shape=[
            jax.ShapeDtypeStruct((B, 3, n), jnp.int32),
            jax.ShapeDtypeStruct((B, 3, n), jnp.float32),
        ],
    )(unknown_t, known)


def _mlp1_kernel(idx_ref, w_ref, kf_ref, uf_ref, W1_ref, b1_ref,
                 h_ref, st_ref, *, m):
    NB = idx_ref.shape[2]
    iota = jax.lax.broadcasted_iota(jnp.int32, (m, NB), 0)
    S = None
    for t in range(3):
        it = idx_ref[0, t].reshape(1, -1)
        wt = w_ref[0, t].reshape(1, -1)
        sel = jnp.where(iota == it, wt, 0.0)
        S = sel if S is None else S + sel
    interp = jnp.dot(kf_ref[0], S, preferred_element_type=jnp.float32)
    nf = jnp.concatenate([interp, uf_ref[0]], axis=0)
    h = jnp.dot(W1_ref[...], nf, preferred_element_type=jnp.float32) + b1_ref[...]
    h_ref[0] = h

    @pl.when(jnp.logical_and(pl.program_id(0) == 0, pl.program_id(1) == 0))
    def _():
        st_ref[...] = jnp.zeros_like(st_ref)

    s0 = jnp.sum(h, axis=1, keepdims=True)
    s1 = jnp.sum(h * h, axis=1, keepdims=True)
    st_ref[...] += jnp.concatenate([s0, s1], axis=1)


def _mlp1(idx3, w3, known_feats, unknow_feats, W1, b1c):
    B, C2, m = known_feats.shape
    C1 = unknow_feats.shape[1]
    n = unknow_feats.shape[2]
    Co = W1.shape[0]
    NB = min(n, 512)
    grid = (B, n // NB)
    return pl.pallas_call(
        functools.partial(_mlp1_kernel, m=m),
        grid=grid,
        in_specs=[
            pl.BlockSpec((1, 3, NB), lambda b, i: (b, 0, i)),
            pl.BlockSpec((1, 3, NB), lambda b, i: (b, 0, i)),
            pl.BlockSpec((1, C2, m), lambda b, i: (b, 0, 0)),
            pl.BlockSpec((1, C1, NB), lambda b, i: (b, 0, i)),
            pl.BlockSpec((Co, C2 + C1), lambda b, i: (0, 0)),
            pl.BlockSpec((Co, 1), lambda b, i: (0, 0)),
        ],
        out_specs=[
            pl.BlockSpec((1, Co, NB), lambda b, i: (b, 0, i)),
            pl.BlockSpec((Co, 2), lambda b, i: (0, 0)),
        ],
        out_shape=[
            jax.ShapeDtypeStruct((B, Co, n), jnp.float32),
            jax.ShapeDtypeStruct((Co, 2), jnp.float32),
        ],
    )(idx3, w3, known_feats, unknow_feats, W1, b1c)


def _mlp2_kernel(h_ref, st_ref, g_ref, be_ref, W_ref, b_ref,
                 o_ref, st2_ref, *, count):
    st = st_ref[...]
    mean = st[:, 0:1] * (1.0 / count)
    var = st[:, 1:2] * (1.0 / count) - mean * mean
    scale = g_ref[...] * jax.lax.rsqrt(var + 1e-5)
    x = jnp.maximum((h_ref[0] - mean) * scale + be_ref[...], 0.0)
    h2 = jnp.dot(W_ref[...], x, preferred_element_type=jnp.float32) + b_ref[...]
    o_ref[0] = h2

    @pl.when(jnp.logical_and(pl.program_id(0) == 0, pl.program_id(1) == 0))
    def _():
        st2_ref[...] = jnp.zeros_like(st2_ref)

    s0 = jnp.sum(h2, axis=1, keepdims=True)
    s1 = jnp.sum(h2 * h2, axis=1, keepdims=True)
    st2_ref[...] += jnp.concatenate([s0, s1], axis=1)


def _mlp2(h1, st1, g1c, be1c, W2, b2c, count):
    B, Ci, n = h1.shape
    Co = W2.shape[0]
    NB = min(n, 1024)
    grid = (B, n // NB)
    return pl.pallas_call(
        functools.partial(_mlp2_kernel, count=count),
        grid=grid,
        in_specs=[
            pl.BlockSpec((1, Ci, NB), lambda b, i: (b, 0, i)),
            pl.BlockSpec((Ci, 2), lambda b, i: (0, 0)),
            pl.BlockSpec((Ci, 1), lambda b, i: (0, 0)),
            pl.BlockSpec((Ci, 1), lambda b, i: (0, 0)),
            pl.BlockSpec((Co, Ci), lambda b, i: (0, 0)),
            pl.BlockSpec((Co, 1), lambda b, i: (0, 0)),
        ],
        out_specs=[
            pl.BlockSpec((1, Co, NB), lambda b, i: (b, 0, i)),
            pl.BlockSpec((Co, 2), lambda b, i: (0, 0)),
        ],
        out_shape=[
            jax.ShapeDtypeStruct((B, Co, n), jnp.float32),
            jax.ShapeDtypeStruct((Co, 2), jnp.float32),
        ],
    )(h1, st1, g1c, be1c, W2, b2c)


def _bn3_kernel(h_ref, st_ref, g_ref, be_ref, o_ref, *, count):
    st = st_ref[...]
    mean = st[:, 0:1] * (1.0 / count)
    var = st[:, 1:2] * (1.0 / count) - mean * mean
    scale = g_ref[...] * jax.lax.rsqrt(var + 1e-5)
    o_ref[0] = jnp.maximum((h_ref[0] - mean) * scale + be_ref[...], 0.0)


def _bn3(h2, st2, g2c, be2c, count):
    B, C, n = h2.shape
    NB = min(n, 1024)
    grid = (B, n // NB)
    return pl.pallas_call(
        functools.partial(_bn3_kernel, count=count),
        grid=grid,
        in_specs=[
            pl.BlockSpec((1, C, NB), lambda b, i: (b, 0, i)),
            pl.BlockSpec((C, 2), lambda b, i: (0, 0)),
            pl.BlockSpec((C, 1), lambda b, i: (0, 0)),
            pl.BlockSpec((C, 1), lambda b, i: (0, 0)),
        ],
        out_specs=pl.BlockSpec((1, C, NB), lambda b, i: (b, 0, i)),
        out_shape=jax.ShapeDtypeStruct((B, C, n), jnp.float32),
    )(h2, st2, g2c, be2c)


def kernel(unknown, known, unknow_feats, known_feats,
           W1, b1, g1, be1, W2, b2, g2, be2):
    B, n, _ = unknown.shape
    Co1 = W1.shape[0]
    Co2 = W2.shape[0]
    count = float(B * n)

    unknown_t = jnp.transpose(unknown, (0, 2, 1))
    idx3, w3 = _three_nn(unknown_t, known)

    h1, st1 = _mlp1(idx3, w3, known_feats, unknow_feats,
                    W1, b1.reshape(Co1, 1))
    h2, st2 = _mlp2(h1, st1, g1.reshape(Co1, 1), be1.reshape(Co1, 1),
                    W2, b2.reshape(Co2, 1), count)
    out = _bn3(h2, st2, g2.reshape(Co2, 1), be2.reshape(Co2, 1), count)
    return out


# speedup vs baseline: 22.6681x; 22.6681x over previous
import functools

import jax
import jax.numpy as jnp
from jax.experimental import pallas as pl

_INF = 3.0e38


def _nn_kernel(unk_ref, kn_ref, idx_ref, w_ref, *, m):
    u = unk_ref[0]
    kn = kn_ref[0]
    d2 = None
    for c in range(3):
        kc = kn[:, c:c + 1]
        uc = u[c].reshape(1, -1)
        diff = kc - uc
        sq = diff * diff
        d2 = sq if d2 is None else d2 + sq
    iota = jax.lax.broadcasted_iota(jnp.int32, d2.shape, 0)
    d = d2
    idxs, dists = [], []
    for _ in range(3):
        mn = jnp.min(d, axis=0, keepdims=True)
        am = jnp.min(jnp.where(d == mn, iota, m), axis=0, keepdims=True)
        d = jnp.where(iota == am, _INF, d)
        idxs.append(am)
        dists.append(jnp.sqrt(jnp.maximum(mn, 0.0)))
    idx_ref[0] = jnp.concatenate(idxs, axis=0)
    dr = [1.0 / (dd + 1e-8) for dd in dists]
    norm = dr[0] + dr[1] + dr[2]
    w_ref[0] = jnp.concatenate([x / norm for x in dr], axis=0)


def _three_nn(unknown_t, known):
    B, _, n = unknown_t.shape
    m = known.shape[1]
    NB = min(n, 2048)
    grid = (B, n // NB)
    return pl.pallas_call(
        functools.partial(_nn_kernel, m=m),
        grid=grid,
        in_specs=[
            pl.BlockSpec((1, 3, NB), lambda b, i: (b, 0, i)),
            pl.BlockSpec((1, m, 3), lambda b, i: (b, 0, 0)),
        ],
        out_specs=[
            pl.BlockSpec((1, 3, NB), lambda b, i: (b, 0, i)),
            pl.BlockSpec((1, 3, NB), lambda b, i: (b, 0, i)),
        ],
        out_shape=[
            jax.ShapeDtypeStruct((B, 3, n), jnp.int32),
            jax.ShapeDtypeStruct((B, 3, n), jnp.float32),
        ],
    )(unknown_t, known)


def _mlp1_kernel(idx_ref, w_ref, kf_ref, uf_ref, W1_ref, b1_ref,
                 h_ref, st_ref, *, m):
    NB = idx_ref.shape[2]
    iota = jax.lax.broadcasted_iota(jnp.int32, (m, NB), 0)
    S = None
    for t in range(3):
        it = idx_ref[0, t].reshape(1, -1)
        wt = w_ref[0, t].reshape(1, -1)
        sel = jnp.where(iota == it, wt, 0.0)
        S = sel if S is None else S + sel
    interp = jnp.dot(kf_ref[0], S, preferred_element_type=jnp.float32)
    nf = jnp.concatenate([interp, uf_ref[0]], axis=0)
    h = jnp.dot(W1_ref[...], nf, preferred_element_type=jnp.float32) + b1_ref[...]
    h_ref[0] = h

    @pl.when(jnp.logical_and(pl.program_id(0) == 0, pl.program_id(1) == 0))
    def _():
        st_ref[...] = jnp.zeros_like(st_ref)

    s0 = jnp.sum(h, axis=1, keepdims=True)
    s1 = jnp.sum(h * h, axis=1, keepdims=True)
    st_ref[...] += jnp.concatenate([s0, s1], axis=1)


def _mlp1(idx3, w3, known_feats, unknow_feats, W1, b1c):
    B, C2, m = known_feats.shape
    C1 = unknow_feats.shape[1]
    n = unknow_feats.shape[2]
    Co = W1.shape[0]
    NB = min(n, 512)
    grid = (B, n // NB)
    return pl.pallas_call(
        functools.partial(_mlp1_kernel, m=m),
        grid=grid,
        in_specs=[
            pl.BlockSpec((1, 3, NB), lambda b, i: (b, 0, i)),
            pl.BlockSpec((1, 3, NB), lambda b, i: (b, 0, i)),
            pl.BlockSpec((1, C2, m), lambda b, i: (b, 0, 0)),
            pl.BlockSpec((1, C1, NB), lambda b, i: (b, 0, i)),
            pl.BlockSpec((Co, C2 + C1), lambda b, i: (0, 0)),
            pl.BlockSpec((Co, 1), lambda b, i: (0, 0)),
        ],
        out_specs=[
            pl.BlockSpec((1, Co, NB), lambda b, i: (b, 0, i)),
            pl.BlockSpec((Co, 2), lambda b, i: (0, 0)),
        ],
        out_shape=[
            jax.ShapeDtypeStruct((B, Co, n), jnp.float32),
            jax.ShapeDtypeStruct((Co, 2), jnp.float32),
        ],
    )(idx3, w3, known_feats, unknow_feats, W1, b1c)


def _mlp2_kernel(h_ref, st_ref, g_ref, be_ref, W_ref, b_ref,
                 o_ref, st2_ref, *, count):
    st = st_ref[...]
    mean = st[:, 0:1] * (1.0 / count)
    var = st[:, 1:2] * (1.0 / count) - mean * mean
    scale = g_ref[...] * jax.lax.rsqrt(var + 1e-5)
    x = jnp.maximum((h_ref[0] - mean) * scale + be_ref[...], 0.0)
    h2 = jnp.dot(W_ref[...], x, preferred_element_type=jnp.float32) + b_ref[...]
    o_ref[0] = h2

    @pl.when(jnp.logical_and(pl.program_id(0) == 0, pl.program_id(1) == 0))
    def _():
        st2_ref[...] = jnp.zeros_like(st2_ref)

    s0 = jnp.sum(h2, axis=1, keepdims=True)
    s1 = jnp.sum(h2 * h2, axis=1, keepdims=True)
    st2_ref[...] += jnp.concatenate([s0, s1], axis=1)


def _mlp2(h1, st1, g1c, be1c, W2, b2c, count):
    B, Ci, n = h1.shape
    Co = W2.shape[0]
    NB = min(n, 1024)
    grid = (B, n // NB)
    return pl.pallas_call(
        functools.partial(_mlp2_kernel, count=count),
        grid=grid,
        in_specs=[
            pl.BlockSpec((1, Ci, NB), lambda b, i: (b, 0, i)),
            pl.BlockSpec((Ci, 2), lambda b, i: (0, 0)),
            pl.BlockSpec((Ci, 1), lambda b, i: (0, 0)),
            pl.BlockSpec((Ci, 1), lambda b, i: (0, 0)),
            pl.BlockSpec((Co, Ci), lambda b, i: (0, 0)),
            pl.BlockSpec((Co, 1), lambda b, i: (0, 0)),
        ],
        out_specs=[
            pl.BlockSpec((1, Co, NB), lambda b, i: (b, 0, i)),
            pl.BlockSpec((Co, 2), lambda b, i: (0, 0)),
        ],
        out_shape=[
            jax.ShapeDtypeStruct((B, Co, n), jnp.float32),
            jax.ShapeDtypeStruct((Co, 2), jnp.float32),
        ],
    )(h1, st1, g1c, be1c, W2, b2c)


def _bn3_kernel(h_ref, st_ref, g_ref, be_ref, o_ref, *, count):
    st = st_ref[...]
    mean = st[:, 0:1] * (1.0 / count)
    var = st[:, 1:2] * (1.0 / count) - mean * mean
    scale = g_ref[...] * jax.lax.rsqrt(var + 1e-5)
    o_ref[0] = jnp.maximum((h_ref[0] - mean) * scale + be_ref[...], 0.0)


def _bn3(h2, st2, g2c, be2c, count):
    B, C, n = h2.shape
    NB = min(n, 1024)
    grid = (B, n // NB)
    return pl.pallas_call(
        functools.partial(_bn3_kernel, count=count),
        grid=grid,
        in_specs=[
            pl.BlockSpec((1, C, NB), lambda b, i: (b, 0, i)),
            pl.BlockSpec((C, 2), lambda b, i: (0, 0)),
            pl.BlockSpec((C, 1), lambda b, i: (0, 0)),
            pl.BlockSpec((C, 1), lambda b, i: (0, 0)),
        ],
        out_specs=pl.BlockSpec((1, C, NB), lambda b, i: (b, 0, i)),
        out_shape=jax.ShapeDtypeStruct((B, C, n), jnp.float32),
    )(h2, st2, g2c, be2c)


def kernel(unknown, known, unknow_feats, known_feats,
           W1, b1, g1, be1, W2, b2, g2, be2):
    B, n, _ = unknown.shape
    Co1 = W1.shape[0]
    Co2 = W2.shape[0]
    count = float(B * n)

    unknown_t = jnp.transpose(unknown, (0, 2, 1))
    idx3, w3 = _three_nn(unknown_t, known)

    h1, st1 = _mlp1(idx3, w3, known_feats, unknow_feats,
                    W1, b1.reshape(Co1, 1))
    h2, st2 = _mlp2(h1, st1, g1.reshape(Co1, 1), be1.reshape(Co1, 1),
                    W2, b2.reshape(Co2, 1), count)
    out = _bn3(h2, st2, g2.reshape(Co2, 1), be2.reshape(Co2, 1), count)
    return out
